# baseline (device time: 63795 ns/iter reference)
import jax
import jax.numpy as jnp
from jax import lax
from jax.experimental import pallas as pl
from jax.experimental.pallas import tpu as pltpu

N_DEV = 8
N_HOPS = N_DEV - 1
E_LOCAL = 4


def kernel(x, router_W, route_idx, expert_W, shared_W):
    n, d = x.shape
    h = shared_W.shape[1]

    def body(x_ref, rw_ref, idx_ref, ew_ref, sw_ref, out_ref,
             comm_ref, send_sems, recv_sems):
        my = lax.axis_index("i")
        left = lax.rem(my + N_DEV - 1, N_DEV)
        right = lax.rem(my + 1, N_DEV)

        barrier = pltpu.get_barrier_semaphore()
        for nbr in (left, right):
            pl.semaphore_signal(
                barrier, inc=1,
                device_id=(nbr,), device_id_type=pl.DeviceIdType.MESH,
            )
        pl.semaphore_wait(barrier, 2)

        xf = x_ref[...]
        xbf = xf.astype(jnp.bfloat16)

        scores = jnp.dot(xf, rw_ref[...], preferred_element_type=jnp.float32)
        s_max = jnp.max(scores, axis=-1, keepdims=True)
        e = jnp.exp(scores - s_max)
        probs = e / jnp.sum(e, axis=-1, keepdims=True)
        ridx = idx_ref[...]
        col = lax.broadcasted_iota(jnp.int32, probs.shape, 1)
        p_top = jnp.sum(jnp.where(col == ridx, probs, 0.0),
                        axis=-1, keepdims=True)

        partial = jnp.zeros((n, h), jnp.float32)
        for j in range(E_LOCAL):
            e_glob = my * E_LOCAL + j
            gate = jnp.where(ridx == e_glob, p_top, 0.0)
            y = jnp.dot(xbf, ew_ref[j].astype(jnp.bfloat16),
                        preferred_element_type=jnp.float32)
            partial = partial + gate * y

        shared = jnp.dot(xbf, sw_ref[...].astype(jnp.bfloat16),
                         preferred_element_type=jnp.float32)

        out_ref[...] = shared + partial
        comm_ref[0] = partial.astype(jnp.bfloat16)

        for hp in range(N_HOPS):
            rdma = pltpu.make_async_remote_copy(
                src_ref=comm_ref.at[hp],
                dst_ref=comm_ref.at[hp + 1],
                send_sem=send_sems.at[hp],
                recv_sem=recv_sems.at[hp],
                device_id=(right,),
                device_id_type=pl.DeviceIdType.MESH,
            )
            rdma.start()
            rdma.wait()
            out_ref[...] += comm_ref[hp + 1].astype(jnp.float32)

    return pl.pallas_call(
        body,
        out_shape=jax.ShapeDtypeStruct((n, h), jnp.float32),
        in_specs=[pl.BlockSpec(memory_space=pltpu.VMEM)] * 5,
        out_specs=pl.BlockSpec(memory_space=pltpu.VMEM),
        scratch_shapes=[
            pltpu.VMEM((N_DEV, n, h), jnp.bfloat16),
            pltpu.SemaphoreType.DMA((N_HOPS,)),
            pltpu.SemaphoreType.DMA((N_HOPS,)),
        ],
        compiler_params=pltpu.CompilerParams(collective_id=0),
    )(x, router_W, route_idx, expert_W, shared_W)


# device time: 32800 ns/iter; 1.9450x vs baseline; 1.9450x over previous
import jax
import jax.numpy as jnp
from jax import lax
from jax.experimental import pallas as pl
from jax.experimental.pallas import tpu as pltpu

N_DEV = 8
E_LOCAL = 4

_RS_STEPS = ((1, 256), (4, 128), (2, 64))
_AG_STEPS = ((2, 64), (4, 128), (1, 256))


def kernel(x, router_W, route_idx, expert_W, shared_W):
    n, d = x.shape
    h = shared_W.shape[1]

    def body(x_ref, rw_ref, idx_ref, ew_ref, sw_ref, out_ref,
             acc_ref, stage_ref, r1_ref, r2_ref, r3_ref, ag_ref,
             rs_send, rs_recv, ag_send, ag_recv):
        my = lax.axis_index("i")

        barrier = pltpu.get_barrier_semaphore()
        for m in (1, 2, 4):
            pl.semaphore_signal(
                barrier, inc=1,
                device_id=(my ^ m,), device_id_type=pl.DeviceIdType.MESH,
            )
        pl.semaphore_wait(barrier, 3)

        xf = x_ref[...]
        xbf = xf.astype(jnp.bfloat16)

        scores = jnp.dot(xf, rw_ref[...], preferred_element_type=jnp.float32)
        s_max = jnp.max(scores, axis=-1, keepdims=True)
        e = jnp.exp(scores - s_max)
        probs = e / jnp.sum(e, axis=-1, keepdims=True)
        ridx = idx_ref[...]
        col = lax.broadcasted_iota(jnp.int32, probs.shape, 1)
        p_top = jnp.sum(jnp.where(col == ridx, probs, 0.0),
                        axis=-1, keepdims=True)

        partial = jnp.zeros((n, h), jnp.float32)
        for j in range(E_LOCAL):
            e_glob = my * E_LOCAL + j
            gate = jnp.where(ridx == e_glob, p_top, 0.0)
            y = jnp.dot(xbf, ew_ref[j].astype(jnp.bfloat16),
                        preferred_element_type=jnp.float32)
            partial = partial + gate * y
        acc_ref[...] = partial

        out_ref[...] = jnp.dot(xbf, sw_ref[...].astype(jnp.bfloat16),
                               preferred_element_type=jnp.float32)

        recv_refs = (r1_ref, r2_ref, r3_ref)

        lo = jnp.int32(0)
        for k, (m, half) in enumerate(_RS_STEPS):
            bit = (my & m) != 0
            send_off = lo + jnp.where(bit, 0, half)
            keep_off = lo + jnp.where(bit, half, 0)
            stage_ref[pl.ds(0, half), :] = (
                acc_ref[pl.ds(send_off, half), :].astype(jnp.bfloat16))
            rdma = pltpu.make_async_remote_copy(
                src_ref=stage_ref.at[pl.ds(0, half)],
                dst_ref=recv_refs[k],
                send_sem=rs_send.at[k],
                recv_sem=rs_recv.at[k],
                device_id=(my ^ m,),
                device_id_type=pl.DeviceIdType.MESH,
            )
            rdma.start()
            rdma.wait()
            acc_ref[pl.ds(keep_off, half), :] += (
                recv_refs[k][...].astype(jnp.float32))
            lo = keep_off

        seg = n // N_DEV
        ag_ref[pl.ds(lo, seg), :] = acc_ref[pl.ds(lo, seg), :].astype(
            jnp.bfloat16)

        for k, (m, rows) in enumerate(_AG_STEPS):
            rdma = pltpu.make_async_remote_copy(
                src_ref=ag_ref.at[pl.ds(lo, rows)],
                dst_ref=ag_ref.at[pl.ds(lo, rows)],
                send_sem=ag_send.at[k],
                recv_sem=ag_recv.at[k],
                device_id=(my ^ m,),
                device_id_type=pl.DeviceIdType.MESH,
            )
            rdma.start()
            rdma.wait()
            bit = (my & m) != 0
            lo = jnp.where(bit, lo - rows, lo)

        out_ref[...] += ag_ref[...].astype(jnp.float32)

    return pl.pallas_call(
        body,
        out_shape=jax.ShapeDtypeStruct((n, h), jnp.float32),
        in_specs=[pl.BlockSpec(memory_space=pltpu.VMEM)] * 5,
        out_specs=pl.BlockSpec(memory_space=pltpu.VMEM),
        scratch_shapes=[
            pltpu.VMEM((n, h), jnp.float32),
            pltpu.VMEM((n // 2, h), jnp.bfloat16),
            pltpu.VMEM((n // 2, h), jnp.bfloat16),
            pltpu.VMEM((n // 4, h), jnp.bfloat16),
            pltpu.VMEM((n // 8, h), jnp.bfloat16),
            pltpu.VMEM((n, h), jnp.bfloat16),
            pltpu.SemaphoreType.DMA((3,)),
            pltpu.SemaphoreType.DMA((3,)),
            pltpu.SemaphoreType.DMA((3,)),
            pltpu.SemaphoreType.DMA((3,)),
        ],
        compiler_params=pltpu.CompilerParams(collective_id=0),
    )(x, router_W, route_idx, expert_W, shared_W)


# device time: 21846 ns/iter; 2.9202x vs baseline; 1.5014x over previous
import jax
import jax.numpy as jnp
from jax import lax
from jax.experimental import pallas as pl
from jax.experimental.pallas import tpu as pltpu

N_DEV = 8
E_LOCAL = 4


def dsa(off, size):
    return pl.ds(pl.multiple_of(off, 64), size)


def kernel(x, router_W, route_idx, expert_W, shared_W):
    n, d = x.shape
    h = shared_W.shape[1]
    hh = h // 2
    seg = n // N_DEV

    def body(x_ref, rw_ref, idx_ref, ew_ref, sw_ref, out_ref,
             acc_ref, stA_ref, stB_ref, landA_ref, landB_ref,
             segA_ref, segB_ref, agA_ref, agB_ref,
             rsA_send, rsA_recv, rsB_send, rsB_recv,
             agA_send, agA_recv, agB_send, agB_recv):
        my = lax.axis_index("i")

        barrier = pltpu.get_barrier_semaphore()
        for j in range(1, N_DEV):
            pl.semaphore_signal(
                barrier, inc=1,
                device_id=(lax.rem(my + j, N_DEV),),
                device_id_type=pl.DeviceIdType.MESH,
            )
        pl.semaphore_wait(barrier, N_DEV - 1)

        xf = x_ref[...]
        xbf = xf.astype(jnp.bfloat16)

        scores = jnp.dot(xf, rw_ref[...], preferred_element_type=jnp.float32)
        s_max = jnp.max(scores, axis=-1, keepdims=True)
        e = jnp.exp(scores - s_max)
        probs = e / jnp.sum(e, axis=-1, keepdims=True)
        ridx = idx_ref[...]
        col = lax.broadcasted_iota(jnp.int32, probs.shape, 1)
        p_top = jnp.sum(jnp.where(col == ridx, probs, 0.0),
                        axis=-1, keepdims=True)

        def partial_rows(r0, r1):
            xs = xbf[r0:r1]
            acc = jnp.zeros((r1 - r0, h), jnp.float32)
            for j in range(E_LOCAL):
                gate = jnp.where(ridx[r0:r1] == my * E_LOCAL + j,
                                 p_top[r0:r1], 0.0)
                y = jnp.dot(xs, ew_ref[j].astype(jnp.bfloat16),
                            preferred_element_type=jnp.float32)
                acc = acc + gate * y
            return acc

        def sends(stage_ref, land_ref, send_sems, recv_sems):
            rs, targets = [], []
            for j in range(1, N_DEV):
                t = lax.rem(my + j, N_DEV)
                targets.append(t)
                rs.append(pltpu.make_async_remote_copy(
                    src_ref=stage_ref.at[dsa(t * seg, seg)],
                    dst_ref=land_ref.at[j - 1],
                    send_sem=send_sems.at[j - 1],
                    recv_sem=recv_sems.at[j - 1],
                    device_id=(t,),
                    device_id_type=pl.DeviceIdType.MESH,
                ))
            return rs, targets

        rsA, targets = sends(stA_ref, landA_ref, rsA_send, rsA_recv)
        rsB, _ = sends(stB_ref, landB_ref, rsB_send, rsB_recv)

        p1 = partial_rows(0, n // 2)
        acc_ref[: n // 2] = p1
        stA_ref[: n // 2] = p1[:, :hh].astype(jnp.bfloat16)
        stB_ref[: n // 2] = p1[:, hh:].astype(jnp.bfloat16)
        for j in range(1, N_DEV):
            @pl.when(targets[j - 1] < N_DEV // 2)
            def _(j=j):
                rsA[j - 1].start()

        p2 = partial_rows(n // 2, n)
        acc_ref[n // 2:] = p2
        stA_ref[n // 2:] = p2[:, :hh].astype(jnp.bfloat16)
        stB_ref[n // 2:] = p2[:, hh:].astype(jnp.bfloat16)
        for j in range(1, N_DEV):
            @pl.when(targets[j - 1] >= N_DEV // 2)
            def _(j=j):
                rsA[j - 1].start()
        for j in range(1, N_DEV):
            rsB[j - 1].start()

        out_ref[...] = jnp.dot(xbf, sw_ref[...].astype(jnp.bfloat16),
                               preferred_element_type=jnp.float32)

        base = my * seg

        def gather_sends(seg_ref, ag_ref, send_sems, recv_sems):
            ag = []
            for j in range(1, N_DEV):
                t = lax.rem(my + j, N_DEV)
                ag.append(pltpu.make_async_remote_copy(
                    src_ref=seg_ref,
                    dst_ref=ag_ref.at[dsa(base, seg)],
                    send_sem=send_sems.at[j - 1],
                    recv_sem=recv_sems.at[j - 1],
                    device_id=(t,),
                    device_id_type=pl.DeviceIdType.MESH,
                ))
            return ag

        totalA = acc_ref[dsa(base, seg), :hh]
        for j in range(1, N_DEV):
            rsA[j - 1].wait_recv()
            totalA = totalA + landA_ref[j - 1].astype(jnp.float32)
        segA_ref[...] = totalA.astype(jnp.bfloat16)
        agA = gather_sends(segA_ref, agA_ref, agA_send, agA_recv)
        for g in agA:
            g.start()
        out_ref[dsa(base, seg), :hh] += totalA

        totalB = acc_ref[dsa(base, seg), hh:]
        for j in range(1, N_DEV):
            rsB[j - 1].wait_recv()
            totalB = totalB + landB_ref[j - 1].astype(jnp.float32)
        segB_ref[...] = totalB.astype(jnp.bfloat16)
        agB = gather_sends(segB_ref, agB_ref, agB_send, agB_recv)
        for g in agB:
            g.start()
        out_ref[dsa(base, seg), hh:] += totalB

        for j in range(1, N_DEV):
            s = lax.rem(my - j + N_DEV, N_DEV)
            agA[j - 1].wait_recv()
            out_ref[dsa(s * seg, seg), :hh] += agA_ref[
                dsa(s * seg, seg), :].astype(jnp.float32)
        for j in range(1, N_DEV):
            s = lax.rem(my - j + N_DEV, N_DEV)
            agB[j - 1].wait_recv()
            out_ref[dsa(s * seg, seg), hh:] += agB_ref[
                dsa(s * seg, seg), :].astype(jnp.float32)

        for r in rsA + rsB + agA + agB:
            r.wait_send()

    return pl.pallas_call(
        body,
        out_shape=jax.ShapeDtypeStruct((n, h), jnp.float32),
        in_specs=[pl.BlockSpec(memory_space=pltpu.VMEM)] * 5,
        out_specs=pl.BlockSpec(memory_space=pltpu.VMEM),
        scratch_shapes=[
            pltpu.VMEM((n, h), jnp.float32),
            pltpu.VMEM((n, hh), jnp.bfloat16),
            pltpu.VMEM((n, hh), jnp.bfloat16),
            pltpu.VMEM((N_DEV - 1, seg, hh), jnp.bfloat16),
            pltpu.VMEM((N_DEV - 1, seg, hh), jnp.bfloat16),
            pltpu.VMEM((seg, hh), jnp.bfloat16),
            pltpu.VMEM((seg, hh), jnp.bfloat16),
            pltpu.VMEM((n, hh), jnp.bfloat16),
            pltpu.VMEM((n, hh), jnp.bfloat16),
            pltpu.SemaphoreType.DMA((N_DEV - 1,)),
            pltpu.SemaphoreType.DMA((N_DEV - 1,)),
            pltpu.SemaphoreType.DMA((N_DEV - 1,)),
            pltpu.SemaphoreType.DMA((N_DEV - 1,)),
            pltpu.SemaphoreType.DMA((N_DEV - 1,)),
            pltpu.SemaphoreType.DMA((N_DEV - 1,)),
            pltpu.SemaphoreType.DMA((N_DEV - 1,)),
            pltpu.SemaphoreType.DMA((N_DEV - 1,)),
        ],
        compiler_params=pltpu.CompilerParams(collective_id=0),
    )(x, router_W, route_idx, expert_W, shared_W)


# device time: 20974 ns/iter; 3.0416x vs baseline; 1.0416x over previous
import jax
import jax.numpy as jnp
from jax import lax
from jax.experimental import pallas as pl
from jax.experimental.pallas import tpu as pltpu

N_DEV = 8
E_LOCAL = 4
N_STREAMS = 2


def dsa(off, size):
    return pl.ds(pl.multiple_of(off, 64), size)


def kernel(x, router_W, route_idx, expert_W, shared_W):
    n, d = x.shape
    h = shared_W.shape[1]
    hs = h // N_STREAMS
    seg = n // N_DEV

    def body(x_ref, rw_ref, idx_ref, ew_ref, sw_ref, out_ref,
             acc_ref, st_ref, land_ref, seg_ref, ag_ref,
             rs_send, rs_recv, ag_send, ag_recv):
        my = lax.axis_index("i")

        barrier = pltpu.get_barrier_semaphore()
        for j in range(1, N_DEV):
            pl.semaphore_signal(
                barrier, inc=1,
                device_id=(lax.rem(my + j, N_DEV),),
                device_id_type=pl.DeviceIdType.MESH,
            )

        xf = x_ref[...]
        xbf = xf.astype(jnp.bfloat16)

        scores = jnp.dot(xf, rw_ref[...], preferred_element_type=jnp.float32)
        s_max = jnp.max(scores, axis=-1, keepdims=True)
        e = jnp.exp(scores - s_max)
        probs = e / jnp.sum(e, axis=-1, keepdims=True)
        ridx = idx_ref[...]
        col = lax.broadcasted_iota(jnp.int32, probs.shape, 1)
        p_top = jnp.sum(jnp.where(col == ridx, probs, 0.0),
                        axis=-1, keepdims=True)
        gates = [jnp.where(ridx == my * E_LOCAL + j, p_top, 0.0)
                 for j in range(E_LOCAL)]

        def partial_cols(c0):
            acc = jnp.zeros((n, hs), jnp.float32)
            for j in range(E_LOCAL):
                y = jnp.dot(xbf, ew_ref[j, :, c0:c0 + hs].astype(jnp.bfloat16),
                            preferred_element_type=jnp.float32)
                acc = acc + gates[j] * y
            return acc

        base = my * seg

        rs = [[pltpu.make_async_remote_copy(
                   src_ref=st_ref.at[si].at[dsa(lax.rem(my + j, N_DEV) * seg,
                                                seg)],
                   dst_ref=land_ref.at[si].at[j - 1],
                   send_sem=rs_send.at[si, j - 1],
                   recv_sem=rs_recv.at[si, j - 1],
                   device_id=(lax.rem(my + j, N_DEV),),
                   device_id_type=pl.DeviceIdType.MESH)
               for j in range(1, N_DEV)] for si in range(N_STREAMS)]
        ag = [[pltpu.make_async_remote_copy(
                   src_ref=seg_ref.at[si],
                   dst_ref=ag_ref.at[si].at[dsa(base, seg)],
                   send_sem=ag_send.at[si, j - 1],
                   recv_sem=ag_recv.at[si, j - 1],
                   device_id=(lax.rem(my + j, N_DEV),),
                   device_id_type=pl.DeviceIdType.MESH)
               for j in range(1, N_DEV)] for si in range(N_STREAMS)]

        partials = []
        for si in range(N_STREAMS):
            p = partial_cols(si * hs)
            st_ref[si] = p.astype(jnp.bfloat16)
            if si == 0:
                pl.semaphore_wait(barrier, N_DEV - 1)
            for r in rs[si]:
                r.start()
            partials.append(p)
        for si in range(N_STREAMS):
            acc_ref[si] = partials[si]

        shared_seg = jnp.dot(
            x_ref[dsa(base, seg), :].astype(jnp.bfloat16),
            sw_ref[...].astype(jnp.bfloat16),
            preferred_element_type=jnp.float32)

        for si in range(N_STREAMS):
            total = (acc_ref[si, dsa(base, seg), :]
                     + shared_seg[:, si * hs:(si + 1) * hs])
            for j in range(1, N_DEV):
                rs[si][j - 1].wait_recv()
                total = total + land_ref[si, j - 1].astype(jnp.float32)
            seg_ref[si] = total.astype(jnp.bfloat16)
            for g in ag[si]:
                g.start()
            out_ref[dsa(base, seg), si * hs:(si + 1) * hs] = total

        for si in range(N_STREAMS):
            for j in range(1, N_DEV):
                s = lax.rem(my - j + N_DEV, N_DEV)
                ag[si][j - 1].wait_recv()
                out_ref[dsa(s * seg, seg), si * hs:(si + 1) * hs] = (
                    ag_ref[si, dsa(s * seg, seg), :].astype(jnp.float32))

        for group in rs + ag:
            for r in group:
                r.wait_send()

    return pl.pallas_call(
        body,
        out_shape=jax.ShapeDtypeStruct((n, h), jnp.float32),
        in_specs=[pl.BlockSpec(memory_space=pltpu.VMEM)] * 5,
        out_specs=pl.BlockSpec(memory_space=pltpu.VMEM),
        scratch_shapes=[
            pltpu.VMEM((N_STREAMS, n, hs), jnp.float32),
            pltpu.VMEM((N_STREAMS, n, hs), jnp.bfloat16),
            pltpu.VMEM((N_STREAMS, N_DEV - 1, seg, hs), jnp.bfloat16),
            pltpu.VMEM((N_STREAMS, seg, hs), jnp.bfloat16),
            pltpu.VMEM((N_STREAMS, n, hs), jnp.bfloat16),
            pltpu.SemaphoreType.DMA((N_STREAMS, N_DEV - 1)),
            pltpu.SemaphoreType.DMA((N_STREAMS, N_DEV - 1)),
            pltpu.SemaphoreType.DMA((N_STREAMS, N_DEV - 1)),
            pltpu.SemaphoreType.DMA((N_STREAMS, N_DEV - 1)),
        ],
        compiler_params=pltpu.CompilerParams(collective_id=0),
    )(x, router_W, route_idx, expert_W, shared_W)
